# baseline (speedup 1.0000x reference)
import jax
import jax.numpy as jnp
from jax import lax
from jax.experimental import pallas as pl
from jax.experimental.pallas import tpu as pltpu

HIDDEN = 1024
NUM_HEADS = 16
HEAD_DIM = 64
CHUNK = 256
TOPK = 4
S = 2048
C = S // CHUNK
ROPE_BASE = 10000.0
EPS = 1e-6
NEG = -1e30
SCALE = 1.0 / (HEAD_DIM ** 0.5)


def _proj_kernel(hs_ref, hsb_ref, wq_ref, wk_ref, wv_ref, wg1_ref, wg2_ref,
                 cos_ref, sin_ref,
                 q32_ref, qb_ref, kb_ref, vb_ref, g_ref, kmean_ref):
    hs = hs_ref[...]
    f32 = jnp.float32
    bf16 = jnp.bfloat16
    q = jnp.dot(hs, wq_ref[...], preferred_element_type=f32)
    k = jnp.dot(hs, wk_ref[...], preferred_element_type=f32)
    hsb = hsb_ref[...]
    v = jnp.dot(hsb, wv_ref[...], preferred_element_type=f32)
    g = jnp.dot(jnp.dot(hsb, wg1_ref[...], preferred_element_type=f32
                        ).astype(bf16),
                wg2_ref[...], preferred_element_type=f32)
    cos = cos_ref[...]
    sin = sin_ref[...]
    lane = lax.broadcasted_iota(jnp.int32, (CHUNK, HIDDEN), 1)
    first_half = (lane % HEAD_DIM) < (HEAD_DIM // 2)

    def rope(x):
        rot = jnp.where(first_half,
                        -jnp.roll(x, -HEAD_DIM // 2, axis=1),
                        jnp.roll(x, HEAD_DIM // 2, axis=1))
        return x * cos + rot * sin

    q = rope(q)
    k = rope(k)
    q32_ref[...] = q
    qb_ref[...] = (q * SCALE).astype(bf16)
    kb_ref[...] = k.astype(bf16)
    vb_ref[...] = v.astype(bf16)
    g_ref[...] = g
    kmean_ref[...] = jnp.mean(k, axis=0).reshape(1, 1, HIDDEN)


def _attn_kernel(q32_ref, qb_ref, kb_ref, vb_ref, km_ref, g_ref, o_ref):
    c = pl.program_id(1)
    f32 = jnp.float32
    bf16 = jnp.bfloat16
    col = lax.broadcasted_iota(jnp.int32, (CHUNK, C), 1)
    tcol = lax.broadcasted_iota(jnp.int32, (CHUNK, S), 1)
    rowid = lax.broadcasted_iota(jnp.int32, (CHUNK, S), 0)
    causal_neg = jnp.where(tcol <= c * CHUNK + rowid, 0.0, NEG)
    bcol = lax.broadcasted_iota(jnp.int32, (C, S), 1)
    brow = lax.broadcasted_iota(jnp.int32, (C, S), 0)
    expand = (bcol // CHUNK == brow).astype(f32)

    for sub in range(2):
        lo = sub * HEAD_DIM
        hi = lo + HEAD_DIM
        q32 = q32_ref[:, lo:hi]
        km = km_ref[:, lo:hi]
        gate = lax.dot_general(q32, km, (((1,), (1,)), ((), ())),
                               preferred_element_type=f32)
        gate = jnp.where(col > c, -jnp.inf, gate)
        gate = jnp.where(col == c, jnp.inf, gate)
        thresh = jnp.full((CHUNK, 1), -jnp.inf, f32)
        for j in range(C):
            gj = gate[:, j:j + 1]
            cnt = jnp.sum((gate >= gj).astype(f32), axis=1, keepdims=True)
            thresh = jnp.maximum(thresh,
                                 jnp.where(cnt >= TOPK, gj, -jnp.inf))
        sel = (gate >= thresh) & (gate > -jnp.inf)
        selneg = jnp.where(sel, 0.0, NEG)

        s = lax.dot_general(qb_ref[:, lo:hi], kb_ref[:, lo:hi],
                            (((1,), (1,)), ((), ())),
                            preferred_element_type=f32)
        s = s + jnp.dot(selneg, expand, preferred_element_type=f32)
        s = s + causal_neg
        m = jnp.max(s, axis=1, keepdims=True)
        p = jnp.exp(s - m)
        l = jnp.sum(p, axis=1, keepdims=True)
        o = jnp.dot(p.astype(bf16), vb_ref[:, lo:hi],
                    preferred_element_type=f32) / l
        rms = o * lax.rsqrt(jnp.mean(o * o, axis=1, keepdims=True) + EPS)
        o_ref[:, lo:hi] = (rms * jax.nn.sigmoid(g_ref[:, lo:hi])
                           ).astype(bf16)


def _out_kernel(x_ref, wo_ref, out_ref):
    out_ref[...] = jnp.dot(x_ref[...], wo_ref[...],
                           preferred_element_type=jnp.float32)


def kernel(hidden_states, Wq, Wk, Wv, Wo, Wg1, Wg2, o_norm_weight):
    f32 = jnp.float32
    bf16 = jnp.bfloat16
    hs = hidden_states.reshape(S, HIDDEN)
    hsb = hs.astype(bf16)

    d = jnp.arange(HIDDEN)
    fidx = (d % HEAD_DIM) % (HEAD_DIM // 2)
    inv_freq = 1.0 / (ROPE_BASE ** (2.0 * fidx.astype(f32) / HEAD_DIM))
    t = jnp.arange(S, dtype=f32)
    ang = t[:, None] * inv_freq[None, :]
    cos = jnp.cos(ang)
    sin = jnp.sin(ang)

    n_chunks = C
    q32, qb, kb, vb, g, kmean3 = pl.pallas_call(
        _proj_kernel,
        grid=(n_chunks,),
        in_specs=[
            pl.BlockSpec((CHUNK, HIDDEN), lambda c: (c, 0)),
            pl.BlockSpec((CHUNK, HIDDEN), lambda c: (c, 0)),
            pl.BlockSpec((HIDDEN, HIDDEN), lambda c: (0, 0)),
            pl.BlockSpec((HIDDEN, HIDDEN), lambda c: (0, 0)),
            pl.BlockSpec((HIDDEN, HIDDEN), lambda c: (0, 0)),
            pl.BlockSpec((HIDDEN, HEAD_DIM), lambda c: (0, 0)),
            pl.BlockSpec((HEAD_DIM, HIDDEN), lambda c: (0, 0)),
            pl.BlockSpec((CHUNK, HIDDEN), lambda c: (c, 0)),
            pl.BlockSpec((CHUNK, HIDDEN), lambda c: (c, 0)),
        ],
        out_specs=[
            pl.BlockSpec((CHUNK, HIDDEN), lambda c: (c, 0)),
            pl.BlockSpec((CHUNK, HIDDEN), lambda c: (c, 0)),
            pl.BlockSpec((CHUNK, HIDDEN), lambda c: (c, 0)),
            pl.BlockSpec((CHUNK, HIDDEN), lambda c: (c, 0)),
            pl.BlockSpec((CHUNK, HIDDEN), lambda c: (c, 0)),
            pl.BlockSpec((1, 1, HIDDEN), lambda c: (c, 0, 0)),
        ],
        out_shape=[
            jax.ShapeDtypeStruct((S, HIDDEN), f32),
            jax.ShapeDtypeStruct((S, HIDDEN), bf16),
            jax.ShapeDtypeStruct((S, HIDDEN), bf16),
            jax.ShapeDtypeStruct((S, HIDDEN), bf16),
            jax.ShapeDtypeStruct((S, HIDDEN), f32),
            jax.ShapeDtypeStruct((n_chunks, 1, HIDDEN), f32),
        ],
        compiler_params=pltpu.CompilerParams(
            dimension_semantics=("parallel",)),
    )(hs, hsb, Wq, Wk, Wv.astype(bf16), Wg1.astype(bf16), Wg2.astype(bf16),
      cos, sin)
    kmean = kmean3.reshape(n_chunks, HIDDEN)

    n_pairs = NUM_HEADS // 2
    opart = pl.pallas_call(
        _attn_kernel,
        grid=(n_pairs, n_chunks),
        in_specs=[
            pl.BlockSpec((CHUNK, 2 * HEAD_DIM), lambda p, c: (c, p)),
            pl.BlockSpec((CHUNK, 2 * HEAD_DIM), lambda p, c: (c, p)),
            pl.BlockSpec((S, 2 * HEAD_DIM), lambda p, c: (0, p)),
            pl.BlockSpec((S, 2 * HEAD_DIM), lambda p, c: (0, p)),
            pl.BlockSpec((n_chunks, 2 * HEAD_DIM), lambda p, c: (0, p)),
            pl.BlockSpec((CHUNK, 2 * HEAD_DIM), lambda p, c: (c, p)),
        ],
        out_specs=pl.BlockSpec((CHUNK, 2 * HEAD_DIM), lambda p, c: (c, p)),
        out_shape=jax.ShapeDtypeStruct((S, HIDDEN), bf16),
        compiler_params=pltpu.CompilerParams(
            dimension_semantics=("parallel", "arbitrary")),
    )(q32, qb, kb, vb, kmean, g)

    wo_scaled = (jnp.tile(o_norm_weight, NUM_HEADS)[:, None] * Wo
                 ).astype(bf16)
    out = pl.pallas_call(
        _out_kernel,
        grid=(n_chunks,),
        in_specs=[
            pl.BlockSpec((CHUNK, HIDDEN), lambda c: (c, 0)),
            pl.BlockSpec((HIDDEN, HIDDEN), lambda c: (0, 0)),
        ],
        out_specs=pl.BlockSpec((CHUNK, HIDDEN), lambda c: (c, 0)),
        out_shape=jax.ShapeDtypeStruct((S, HIDDEN), f32),
        compiler_params=pltpu.CompilerParams(
            dimension_semantics=("parallel",)),
    )(opart, wo_scaled)
    return out.reshape(1, S, HIDDEN)


# speedup vs baseline: 1.6225x; 1.6225x over previous
import jax
import jax.numpy as jnp
from jax import lax
from jax.experimental import pallas as pl
from jax.experimental.pallas import tpu as pltpu

HIDDEN = 1024
NUM_HEADS = 16
HEAD_DIM = 64
CHUNK = 256
TOPK = 4
S = 2048
C = S // CHUNK
ROPE_BASE = 10000.0
EPS = 1e-6
NEG = -1e30
SCALE = 1.0 / (HEAD_DIM ** 0.5)


def _proj_kernel(hs_ref, hsb_ref, wq_ref, wk_ref, wv_ref, wg1_ref, wg2_ref,
                 cos_ref, sin_ref,
                 q32_ref, qb_ref, kb_ref, vb_ref, g_ref, kmean_ref):
    hs = hs_ref[...]
    f32 = jnp.float32
    bf16 = jnp.bfloat16
    q = jnp.dot(hs, wq_ref[...], preferred_element_type=f32)
    k = jnp.dot(hs, wk_ref[...], preferred_element_type=f32)
    hsb = hsb_ref[...]
    v = jnp.dot(hsb, wv_ref[...], preferred_element_type=f32)
    g = jnp.dot(jnp.dot(hsb, wg1_ref[...], preferred_element_type=f32
                        ).astype(bf16),
                wg2_ref[...], preferred_element_type=f32)
    cos = cos_ref[...]
    sin = sin_ref[...]
    lane = lax.broadcasted_iota(jnp.int32, (CHUNK, HIDDEN), 1)
    first_half = (lane % HEAD_DIM) < (HEAD_DIM // 2)

    def rope(x):
        rot = jnp.where(first_half,
                        -jnp.roll(x, -HEAD_DIM // 2, axis=1),
                        jnp.roll(x, HEAD_DIM // 2, axis=1))
        return x * cos + rot * sin

    q = rope(q)
    k = rope(k)
    q32_ref[...] = q
    qb_ref[...] = (q * SCALE).astype(bf16)
    kb_ref[...] = k.astype(bf16)
    vb_ref[...] = v.astype(bf16)
    g_ref[...] = g
    kmean_ref[...] = jnp.mean(k, axis=0).reshape(1, 1, HIDDEN)


def _attn_kernel(q32_ref, qb_ref, kb_ref, vb_ref, km_ref, g_ref, o_ref):
    c = pl.program_id(0)
    f32 = jnp.float32
    bf16 = jnp.bfloat16
    col = lax.broadcasted_iota(jnp.int32, (CHUNK, C), 1)
    rowid = lax.broadcasted_iota(jnp.int32, (CHUNK, CHUNK), 0)
    colid = lax.broadcasted_iota(jnp.int32, (CHUNK, CHUNK), 1)
    tri_neg = jnp.where(colid <= rowid, 0.0, NEG)

    for sub in range(2):
        lo = sub * HEAD_DIM
        hi = lo + HEAD_DIM
        q32 = q32_ref[:, lo:hi]
        km = km_ref[:, lo:hi]
        gate = lax.dot_general(q32, km, (((1,), (1,)), ((), ())),
                               preferred_element_type=f32)
        gate = jnp.where(col > c, -jnp.inf, gate)
        gate = jnp.where(col == c, jnp.inf, gate)
        cnt = jnp.zeros((CHUNK, C), f32)
        for i in range(1, C):
            cnt = cnt + (jnp.roll(gate, -i, axis=1) > gate).astype(f32)
        sel = (cnt < TOPK) & (gate > -jnp.inf)
        selneg = jnp.where(sel, 0.0, NEG)

        s = lax.dot_general(qb_ref[:, lo:hi], kb_ref[:, lo:hi],
                            (((1,), (1,)), ((), ())),
                            preferred_element_type=f32)
        p_parts = []
        l = jnp.zeros((CHUNK, 1), f32)
        for j in range(C):
            diag = jnp.where(j == c, 1.0, 0.0)
            pj = jnp.exp(s[:, j * CHUNK:(j + 1) * CHUNK]
                         + selneg[:, j:j + 1] + tri_neg * diag)
            l = l + jnp.sum(pj, axis=1, keepdims=True)
            p_parts.append(pj.astype(bf16))
        p = jnp.concatenate(p_parts, axis=1)
        o = jnp.dot(p, vb_ref[:, lo:hi], preferred_element_type=f32) / l
        rms = o * lax.rsqrt(jnp.mean(o * o, axis=1, keepdims=True) + EPS)
        o_ref[:, lo:hi] = (rms * jax.nn.sigmoid(g_ref[:, lo:hi])
                           ).astype(bf16)


def _out_kernel(x_ref, wo_ref, out_ref):
    out_ref[...] = jnp.dot(x_ref[...], wo_ref[...],
                           preferred_element_type=jnp.float32)


def kernel(hidden_states, Wq, Wk, Wv, Wo, Wg1, Wg2, o_norm_weight):
    f32 = jnp.float32
    bf16 = jnp.bfloat16
    hs = hidden_states.reshape(S, HIDDEN)
    hsb = hs.astype(bf16)

    d = jnp.arange(HIDDEN)
    fidx = (d % HEAD_DIM) % (HEAD_DIM // 2)
    inv_freq = 1.0 / (ROPE_BASE ** (2.0 * fidx.astype(f32) / HEAD_DIM))
    t = jnp.arange(S, dtype=f32)
    ang = t[:, None] * inv_freq[None, :]
    cos = jnp.cos(ang)
    sin = jnp.sin(ang)

    n_chunks = C
    q32, qb, kb, vb, g, kmean3 = pl.pallas_call(
        _proj_kernel,
        grid=(n_chunks,),
        in_specs=[
            pl.BlockSpec((CHUNK, HIDDEN), lambda c: (c, 0)),
            pl.BlockSpec((CHUNK, HIDDEN), lambda c: (c, 0)),
            pl.BlockSpec((HIDDEN, HIDDEN), lambda c: (0, 0)),
            pl.BlockSpec((HIDDEN, HIDDEN), lambda c: (0, 0)),
            pl.BlockSpec((HIDDEN, HIDDEN), lambda c: (0, 0)),
            pl.BlockSpec((HIDDEN, HEAD_DIM), lambda c: (0, 0)),
            pl.BlockSpec((HEAD_DIM, HIDDEN), lambda c: (0, 0)),
            pl.BlockSpec((CHUNK, HIDDEN), lambda c: (c, 0)),
            pl.BlockSpec((CHUNK, HIDDEN), lambda c: (c, 0)),
        ],
        out_specs=[
            pl.BlockSpec((CHUNK, HIDDEN), lambda c: (c, 0)),
            pl.BlockSpec((CHUNK, HIDDEN), lambda c: (c, 0)),
            pl.BlockSpec((CHUNK, HIDDEN), lambda c: (c, 0)),
            pl.BlockSpec((CHUNK, HIDDEN), lambda c: (c, 0)),
            pl.BlockSpec((CHUNK, HIDDEN), lambda c: (c, 0)),
            pl.BlockSpec((1, 1, HIDDEN), lambda c: (c, 0, 0)),
        ],
        out_shape=[
            jax.ShapeDtypeStruct((S, HIDDEN), f32),
            jax.ShapeDtypeStruct((S, HIDDEN), bf16),
            jax.ShapeDtypeStruct((S, HIDDEN), bf16),
            jax.ShapeDtypeStruct((S, HIDDEN), bf16),
            jax.ShapeDtypeStruct((S, HIDDEN), f32),
            jax.ShapeDtypeStruct((n_chunks, 1, HIDDEN), f32),
        ],
        compiler_params=pltpu.CompilerParams(
            dimension_semantics=("parallel",)),
    )(hs, hsb, Wq, Wk, Wv.astype(bf16), Wg1.astype(bf16), Wg2.astype(bf16),
      cos, sin)
    kmean = kmean3.reshape(n_chunks, HIDDEN)

    n_pairs = NUM_HEADS // 2
    opart = pl.pallas_call(
        _attn_kernel,
        grid=(n_chunks, n_pairs),
        in_specs=[
            pl.BlockSpec((CHUNK, 2 * HEAD_DIM), lambda c, p: (c, p)),
            pl.BlockSpec((CHUNK, 2 * HEAD_DIM), lambda c, p: (c, p)),
            pl.BlockSpec((S, 2 * HEAD_DIM), lambda c, p: (0, p)),
            pl.BlockSpec((S, 2 * HEAD_DIM), lambda c, p: (0, p)),
            pl.BlockSpec((n_chunks, 2 * HEAD_DIM), lambda c, p: (0, p)),
            pl.BlockSpec((CHUNK, 2 * HEAD_DIM), lambda c, p: (c, p)),
        ],
        out_specs=pl.BlockSpec((CHUNK, 2 * HEAD_DIM), lambda c, p: (c, p)),
        out_shape=jax.ShapeDtypeStruct((S, HIDDEN), bf16),
        compiler_params=pltpu.CompilerParams(
            dimension_semantics=("parallel", "arbitrary")),
    )(q32, qb, kb, vb, kmean, g)

    wo_scaled = (jnp.tile(o_norm_weight, NUM_HEADS)[:, None] * Wo
                 ).astype(bf16)
    out = pl.pallas_call(
        _out_kernel,
        grid=(n_chunks,),
        in_specs=[
            pl.BlockSpec((CHUNK, HIDDEN), lambda c: (c, 0)),
            pl.BlockSpec((HIDDEN, HIDDEN), lambda c: (0, 0)),
        ],
        out_specs=pl.BlockSpec((CHUNK, HIDDEN), lambda c: (c, 0)),
        out_shape=jax.ShapeDtypeStruct((S, HIDDEN), f32),
        compiler_params=pltpu.CompilerParams(
            dimension_semantics=("parallel",)),
    )(opart, wo_scaled)
    return out.reshape(1, S, HIDDEN)
